# Initial kernel scaffold; baseline (speedup 1.0000x reference)
import jax
import jax.numpy as jnp
from jax.experimental import pallas as pl


def kernel(h, edge_features, edge_index, n_edges, bm_w, bm_b, bu_w, bu_b, bn_g, bn_b, fm_w, fm_b, fu_w, fu_b, fn_g, fn_b):
    raise NotImplementedError("write your pallas kernel here")


# speedup vs baseline: 4370.8638x; 4370.8638x over previous
import functools

import jax
import jax.numpy as jnp
from jax import lax
from jax.experimental import pallas as pl
from jax.experimental.pallas import tpu as pltpu
from jax.experimental.pallas import tpu_sc as plsc

_B, _N, _E, _H, _ED = 4, 10000, 160000, 128, 2
_CHUNK = 128
_NTILES = 16
_NCORES = 2
_ROWS_SP = 10112
_STRIPE = _ROWS_SP // _NTILES
_CPB = _E // _CHUNK
_GARBAGE = _N


@functools.cache
def _make_edge_pass(is_back):
    mesh = plsc.VectorSubcoreMesh(core_axis_name="c", subcore_axis_name="s")

    @functools.partial(
        pl.kernel,
        out_type=(jax.ShapeDtypeStruct((_B, _ROWS_SP, _H), jnp.float32),
                  jax.ShapeDtypeStruct((_B * _ROWS_SP,), jnp.float32)),
        mesh=mesh,
        scratch_types=[
            pltpu.VMEM((_CHUNK,), jnp.int32),
            pltpu.VMEM((_CHUNK,), jnp.int32),
            pltpu.VMEM((_CHUNK,), jnp.float32),
            pltpu.VMEM((_CHUNK,), jnp.float32),
            pltpu.VMEM((_CHUNK, _H), jnp.float32),
            pltpu.VMEM((_CHUNK, _H), jnp.float32),
            pltpu.VMEM((_ED, _H), jnp.float32),
            pltpu.VMEM((16 * _B,), jnp.int32),
            pltpu.VMEM((_CHUNK,), jnp.float32),
            pltpu.VMEM((_STRIPE + 8,), jnp.float32),
            pltpu.VMEM((_STRIPE,), jnp.float32),
            pltpu.VMEM_SHARED((_ROWS_SP, _H), jnp.float32),
            pltpu.VMEM_SHARED((_ROWS_SP,), jnp.float32),
            pltpu.SemaphoreType.DMA,
        ],
    )
    def edge_pass(hw_hbm, src_hbm, dst_hbm, ef0_hbm, ef1_hbm, ne_hbm, w01_hbm,
                  agg_hbm, cnt_hbm,
                  src_v, dst_v, ef0_v, ef1_v, grow_v, msg_v, w_v, ne_v,
                  ones_v, zline_v, cstage_v, agg_sp, cnt_sp, sem):
        cid = lax.axis_index("c")
        sid = lax.axis_index("s")
        pltpu.sync_copy(ne_hbm, ne_v)
        pltpu.sync_copy(w01_hbm, w_v)
        w0 = [w_v[0, pl.ds(16 * k, 16)] for k in range(8)]
        w1 = [w_v[1, pl.ds(16 * k, 16)] for k in range(8)]
        iota = lax.broadcasted_iota(jnp.int32, (16,), 0)
        zf = jnp.zeros((16,), jnp.float32)
        onef = jnp.full((16,), 1.0, jnp.float32)
        n_c = jnp.where(sid < (_CPB % _NTILES),
                        _CPB // _NTILES + 1, _CPB // _NTILES)
        for k in range(_CHUNK // 16):
            ones_v[pl.ds(16 * k, 16)] = onef
        def zl(r, c):
            zline_v[pl.ds(16 * r, 16)] = zf
            return c
        lax.fori_loop(0, (_STRIPE + 8) // 16, zl, 0)
        zbase = sid * _STRIPE

        for jj in range(_B // _NCORES):
            b = cid * (_B // _NCORES) + jj
            nev = ne_v[pl.ds(b * 16, 16)]

            def zrow(r, c):
                for k in range(_H // 16):
                    msg_v[r, pl.ds(16 * k, 16)] = zf
                return c
            lax.fori_loop(0, _CHUNK, zrow, 0)
            for part in range(4):
                pltpu.sync_copy(msg_v, agg_sp.at[pl.ds(zbase + part * _CHUNK, _CHUNK)])
            pltpu.sync_copy(msg_v.at[pl.ds(0, _STRIPE - 4 * _CHUNK)],
                            agg_sp.at[pl.ds(zbase + 4 * _CHUNK, _STRIPE - 4 * _CHUNK)])
            pltpu.sync_copy(zline_v.at[pl.ds(0, _STRIPE)],
                            cnt_sp.at[pl.ds(zbase, _STRIPE)])
            plsc.subcore_barrier()

            def chunk_body(c, carry):
                g = c * _NTILES + sid
                e0 = g * _CHUNK
                eb = b * _E + e0
                pltpu.sync_copy(src_hbm.at[pl.ds(eb, _CHUNK)], src_v)
                pltpu.sync_copy(dst_hbm.at[pl.ds(eb, _CHUNK)], dst_v)
                pltpu.sync_copy(ef0_hbm.at[pl.ds(eb, _CHUNK)], ef0_v)
                pltpu.sync_copy(ef1_hbm.at[pl.ds(eb, _CHUNK)], ef1_v)
                for j in range(_CHUNK // 16):
                    s16 = src_v[pl.ds(16 * j, 16)]
                    d16 = dst_v[pl.ds(16 * j, 16)]
                    eid = e0 + 16 * j + iota
                    cmp = (s16 > d16) if is_back else (s16 < d16)
                    m = (eid < nev) & cmp
                    dst_v[pl.ds(16 * j, 16)] = jnp.where(m, d16, _GARBAGE)
                    src_v[pl.ds(16 * j, 16)] = s16 + b * _N
                pltpu.async_copy(hw_hbm.at[src_v], grow_v, sem).wait()

                def egroup(j, cc):
                    ef0g = ef0_v[pl.ds(16 * j, 16)]
                    ef1g = ef1_v[pl.ds(16 * j, 16)]
                    for l in range(16):
                        e = 16 * j + l
                        ef0 = jnp.full((16,), ef0g[l], jnp.float32)
                        ef1 = jnp.full((16,), ef1g[l], jnp.float32)
                        for k in range(8):
                            gk = grow_v[e, pl.ds(16 * k, 16)]
                            msg_v[e, pl.ds(16 * k, 16)] = jnp.maximum(
                                gk + ef0 * w0[k] + ef1 * w1[k], 0.0)
                    return cc
                lax.fori_loop(0, _CHUNK // 16, egroup, 0)
                pltpu.sync_copy(msg_v, agg_sp.at[dst_v], add=True)
                pltpu.sync_copy(ones_v, cnt_sp.at[dst_v], add=True)
                return carry
            lax.fori_loop(0, n_c, chunk_body, 0)
            plsc.subcore_barrier()

            pltpu.sync_copy(agg_sp.at[pl.ds(zbase, _STRIPE)],
                            agg_hbm.at[b, pl.ds(zbase, _STRIPE)])
            pltpu.sync_copy(cnt_sp.at[pl.ds(zbase, _STRIPE)], cstage_v)
            pltpu.sync_copy(cstage_v,
                            cnt_hbm.at[pl.ds(b * _ROWS_SP + zbase, _STRIPE)])
            plsc.subcore_barrier()

    return edge_pass


def _edge_back(*args):
    return _make_edge_pass(True)(*args)


def _edge_fwd(*args):
    return _make_edge_pass(False)(*args)


_BN = 1000


def _premsg_body(h_ref, w_ref, b_ref, o_ref):
    o_ref[0] = jnp.dot(h_ref[0], w_ref[...],
                       preferred_element_type=jnp.float32) + b_ref[...]


def _premsg(h, wt, bias):
    return pl.pallas_call(
        _premsg_body,
        grid=(_B, _N // _BN),
        in_specs=[
            pl.BlockSpec((1, _BN, _H), lambda b, i: (b, i, 0)),
            pl.BlockSpec((_H, _H), lambda b, i: (0, 0)),
            pl.BlockSpec((1, _H), lambda b, i: (0, 0)),
        ],
        out_specs=pl.BlockSpec((1, _BN, _H), lambda b, i: (b, i, 0)),
        out_shape=jax.ShapeDtypeStruct((_B, _N, _H), jnp.float32),
    )(h, wt, bias)


def _norm_update(h, agg, cnt, wuh, wua, bu, g, beta):
    aggn = agg / jnp.maximum(cnt, 1.0)
    u = (jnp.dot(h, wuh, preferred_element_type=jnp.float32)
         + jnp.dot(aggn, wua, preferred_element_type=jnp.float32) + bu)
    x = h + jnp.maximum(u, 0.0)
    mu = jnp.mean(x, axis=-1, keepdims=True)
    xc = x - mu
    var = jnp.mean(xc * xc, axis=-1, keepdims=True)
    return xc * lax.rsqrt(var + 1e-5) * g + beta


def _update_next_body(h_ref, ag_ref, cn_ref, wuh_ref, wua_ref, bu_ref, g_ref,
                      be_ref, wn_ref, bn_ref, oh_ref, ow_ref):
    hn = _norm_update(h_ref[0], ag_ref[0], cn_ref[0], wuh_ref[...],
                      wua_ref[...], bu_ref[...], g_ref[...], be_ref[...])
    oh_ref[0] = hn
    ow_ref[0] = jnp.dot(hn, wn_ref[...],
                        preferred_element_type=jnp.float32) + bn_ref[...]


def _update_last_body(h_ref, ag_ref, cn_ref, wuh_ref, wua_ref, bu_ref, g_ref,
                      be_ref, oh_ref):
    oh_ref[0] = _norm_update(h_ref[0], ag_ref[0], cn_ref[0], wuh_ref[...],
                             wua_ref[...], bu_ref[...], g_ref[...], be_ref[...])


def _specs():
    w = pl.BlockSpec((_H, _H), lambda b, i: (0, 0))
    v = pl.BlockSpec((1, _H), lambda b, i: (0, 0))
    hs = pl.BlockSpec((1, _BN, _H), lambda b, i: (b, i, 0))
    cs = pl.BlockSpec((1, _BN, 1), lambda b, i: (b, i, 0))
    return w, v, hs, cs


def _update_next(h, agg, cnt, wuh, wua, bu, g, beta, wn, bn):
    w, v, hs, cs = _specs()
    return pl.pallas_call(
        _update_next_body,
        grid=(_B, _N // _BN),
        in_specs=[hs, hs, cs, w, w, v, v, v, w, v],
        out_specs=[hs, hs],
        out_shape=[jax.ShapeDtypeStruct((_B, _N, _H), jnp.float32),
                   jax.ShapeDtypeStruct((_B, _N, _H), jnp.float32)],
    )(h, agg, cnt, wuh, wua, bu, g, beta, wn, bn)


def _update_last(h, agg, cnt, wuh, wua, bu, g, beta):
    w, v, hs, cs = _specs()
    return pl.pallas_call(
        _update_last_body,
        grid=(_B, _N // _BN),
        in_specs=[hs, hs, cs, w, w, v, v, v],
        out_specs=hs,
        out_shape=jax.ShapeDtypeStruct((_B, _N, _H), jnp.float32),
    )(h, agg, cnt, wuh, wua, bu, g, beta)


def kernel(h, edge_features, edge_index, n_edges, bm_w, bm_b, bu_w, bu_b,
           bn_g, bn_b, fm_w, fm_b, fu_w, fu_b, fn_g, fn_b):
    src = edge_index[:, 0, :].reshape(_B * _E)
    dst = edge_index[:, 1, :].reshape(_B * _E)
    ef0 = edge_features[:, :, 0].reshape(_B * _E)
    ef1 = edge_features[:, :, 1].reshape(_B * _E)
    ne8 = jnp.repeat(n_edges[:, 0], 16)

    hw0 = _premsg(h, bm_w[:, :_H].T, bm_b.reshape(1, _H))
    agg0, cnt0 = _edge_back(hw0.reshape(_B * _N, _H), src, dst, ef0, ef1,
                            ne8, bm_w[:, _H:].T)
    h1, hw1 = _update_next(h, agg0, cnt0.reshape(_B, _ROWS_SP, 1),
                           bu_w[:, :_H].T, bu_w[:, _H:].T,
                           bu_b.reshape(1, _H), bn_g.reshape(1, _H),
                           bn_b.reshape(1, _H), fm_w[:, :_H].T,
                           fm_b.reshape(1, _H))
    agg1, cnt1 = _edge_fwd(hw1.reshape(_B * _N, _H), src, dst, ef0, ef1,
                           ne8, fm_w[:, _H:].T)
    return _update_last(h1, agg1, cnt1.reshape(_B, _ROWS_SP, 1),
                        fu_w[:, :_H].T, fu_w[:, _H:].T,
                        fu_b.reshape(1, _H), fn_g.reshape(1, _H),
                        fn_b.reshape(1, _H))


# speedup vs baseline: 7147.5320x; 1.6353x over previous
import functools

import jax
import jax.numpy as jnp
from jax import lax
from jax.experimental import pallas as pl
from jax.experimental.pallas import tpu as pltpu
from jax.experimental.pallas import tpu_sc as plsc

_B, _N, _E, _H, _ED = 4, 10000, 160000, 128, 2
_CHUNK = 128
_NTILES = 16
_NCORES = 2
_ROWS_SP = 10112
_STRIPE = _ROWS_SP // _NTILES
_CPB = _E // _CHUNK
_GARBAGE = _N


@functools.cache
def _make_edge_pass(is_back):
    mesh = plsc.VectorSubcoreMesh(core_axis_name="c", subcore_axis_name="s")

    @functools.partial(
        pl.kernel,
        out_type=(jax.ShapeDtypeStruct((_B, _ROWS_SP, _H), jnp.float32),
                  jax.ShapeDtypeStruct((_B * _ROWS_SP,), jnp.float32)),
        mesh=mesh,
        scratch_types=[
            pltpu.VMEM((2 * _CHUNK,), jnp.int32),
            pltpu.VMEM((2 * _CHUNK,), jnp.float32),
            pltpu.VMEM((_CHUNK,), jnp.int32),
            pltpu.VMEM((_CHUNK,), jnp.int32),
            pltpu.VMEM((_CHUNK, _H), jnp.float32),
            pltpu.VMEM((_ED, _H), jnp.float32),
            pltpu.VMEM((16 * _B,), jnp.int32),
            pltpu.VMEM((_CHUNK,), jnp.float32),
            pltpu.VMEM((_STRIPE + 8,), jnp.float32),
            pltpu.VMEM((_STRIPE,), jnp.float32),
            pltpu.VMEM_SHARED((_ROWS_SP, _H), jnp.float32),
            pltpu.VMEM_SHARED((_ROWS_SP,), jnp.float32),
            pltpu.SemaphoreType.DMA,
        ],
    )
    def edge_pass(hw_hbm, meta_hbm, efm_hbm, ne_hbm, w01_hbm,
                  agg_hbm, cnt_hbm,
                  meta_v, efm_v, gidx_v, sidx_v, grow_v, w_v, ne_v,
                  ones_v, zline_v, cstage_v, agg_sp, cnt_sp, sem):
        cid = lax.axis_index("c")
        sid = lax.axis_index("s")
        pltpu.sync_copy(ne_hbm, ne_v)
        pltpu.sync_copy(w01_hbm, w_v)
        w0 = [w_v[0, pl.ds(16 * k, 16)] for k in range(8)]
        w1 = [w_v[1, pl.ds(16 * k, 16)] for k in range(8)]
        iota = lax.broadcasted_iota(jnp.int32, (16,), 0)
        zf = jnp.zeros((16,), jnp.float32)
        onef = jnp.full((16,), 1.0, jnp.float32)
        n_c = jnp.where(sid < (_CPB % _NTILES),
                        _CPB // _NTILES + 1, _CPB // _NTILES)
        for k in range(_CHUNK // 16):
            ones_v[pl.ds(16 * k, 16)] = onef
        def zl(r, c):
            zline_v[pl.ds(16 * r, 16)] = zf
            return c
        lax.fori_loop(0, (_STRIPE + 8) // 16, zl, 0)
        zbase = sid * _STRIPE

        for jj in range(_B // _NCORES):
            b = cid * (_B // _NCORES) + jj
            nev = ne_v[pl.ds(b * 16, 16)]
            g_act = (nev[0] + (_CHUNK - 1)) // _CHUNK
            n_act = jnp.clip((g_act - sid + (_NTILES - 1)) // _NTILES, 0, n_c)

            def zrow(r, c):
                for k in range(_H // 16):
                    grow_v[r, pl.ds(16 * k, 16)] = zf
                return c
            lax.fori_loop(0, _CHUNK, zrow, 0)
            for part in range(4):
                pltpu.sync_copy(grow_v, agg_sp.at[pl.ds(zbase + part * _CHUNK, _CHUNK)])
            pltpu.sync_copy(grow_v.at[pl.ds(0, _STRIPE - 4 * _CHUNK)],
                            agg_sp.at[pl.ds(zbase + 4 * _CHUNK, _STRIPE - 4 * _CHUNK)])
            pltpu.sync_copy(zline_v.at[pl.ds(0, _STRIPE)],
                            cnt_sp.at[pl.ds(zbase, _STRIPE)])
            plsc.subcore_barrier()

            def chunk_body(c, carry):
                g = c * _NTILES + sid
                e0 = g * _CHUNK
                mb = (b * _CPB + g) * (2 * _CHUNK)
                pltpu.sync_copy(meta_hbm.at[pl.ds(mb, 2 * _CHUNK)], meta_v)
                pltpu.sync_copy(efm_hbm.at[pl.ds(mb, 2 * _CHUNK)], efm_v)
                for j in range(_CHUNK // 16):
                    s16 = meta_v[pl.ds(16 * j, 16)]
                    d16 = meta_v[pl.ds(_CHUNK + 16 * j, 16)]
                    eid = e0 + 16 * j + iota
                    cmp = (s16 > d16) if is_back else (s16 < d16)
                    m = (eid < nev) & cmp
                    sidx_v[pl.ds(16 * j, 16)] = jnp.where(m, d16, _GARBAGE)
                    gidx_v[pl.ds(16 * j, 16)] = s16 + b * _N
                pltpu.async_copy(hw_hbm.at[gidx_v], grow_v, sem).wait()

                def egroup(j, cc):
                    ef0g = efm_v[pl.ds(16 * j, 16)]
                    ef1g = efm_v[pl.ds(_CHUNK + 16 * j, 16)]
                    for l in range(16):
                        e = 16 * j + l
                        ef0 = jnp.full((16,), ef0g[l], jnp.float32)
                        ef1 = jnp.full((16,), ef1g[l], jnp.float32)
                        for k in range(8):
                            gk = grow_v[e, pl.ds(16 * k, 16)]
                            grow_v[e, pl.ds(16 * k, 16)] = jnp.maximum(
                                gk + ef0 * w0[k] + ef1 * w1[k], 0.0)
                    return cc
                lax.fori_loop(0, _CHUNK // 16, egroup, 0)
                pltpu.sync_copy(grow_v, agg_sp.at[sidx_v], add=True)
                pltpu.sync_copy(ones_v, cnt_sp.at[sidx_v], add=True)
                return carry
            lax.fori_loop(0, n_act, chunk_body, 0)
            plsc.subcore_barrier()

            pltpu.sync_copy(agg_sp.at[pl.ds(zbase, _STRIPE)],
                            agg_hbm.at[b, pl.ds(zbase, _STRIPE)])
            pltpu.sync_copy(cnt_sp.at[pl.ds(zbase, _STRIPE)], cstage_v)
            pltpu.sync_copy(cstage_v,
                            cnt_hbm.at[pl.ds(b * _ROWS_SP + zbase, _STRIPE)])
            plsc.subcore_barrier()

    return edge_pass


def _edge_back(*args):
    return _make_edge_pass(True)(*args)


def _edge_fwd(*args):
    return _make_edge_pass(False)(*args)


_BN = 1000


def _premsg_body(h_ref, w_ref, b_ref, o_ref):
    o_ref[0] = jnp.dot(h_ref[0], w_ref[...],
                       preferred_element_type=jnp.float32) + b_ref[...]


def _premsg(h, wt, bias):
    return pl.pallas_call(
        _premsg_body,
        grid=(_B, _N // _BN),
        in_specs=[
            pl.BlockSpec((1, _BN, _H), lambda b, i: (b, i, 0)),
            pl.BlockSpec((_H, _H), lambda b, i: (0, 0)),
            pl.BlockSpec((1, _H), lambda b, i: (0, 0)),
        ],
        out_specs=pl.BlockSpec((1, _BN, _H), lambda b, i: (b, i, 0)),
        out_shape=jax.ShapeDtypeStruct((_B, _N, _H), jnp.float32),
    )(h, wt, bias)


def _norm_update(h, agg, cnt, wuh, wua, bu, g, beta):
    aggn = agg / jnp.maximum(cnt, 1.0)
    u = (jnp.dot(h, wuh, preferred_element_type=jnp.float32)
         + jnp.dot(aggn, wua, preferred_element_type=jnp.float32) + bu)
    x = h + jnp.maximum(u, 0.0)
    mu = jnp.mean(x, axis=-1, keepdims=True)
    xc = x - mu
    var = jnp.mean(xc * xc, axis=-1, keepdims=True)
    return xc * lax.rsqrt(var + 1e-5) * g + beta


def _update_next_body(h_ref, ag_ref, cn_ref, wuh_ref, wua_ref, bu_ref, g_ref,
                      be_ref, wn_ref, bn_ref, oh_ref, ow_ref):
    hn = _norm_update(h_ref[0], ag_ref[0], cn_ref[0], wuh_ref[...],
                      wua_ref[...], bu_ref[...], g_ref[...], be_ref[...])
    oh_ref[0] = hn
    ow_ref[0] = jnp.dot(hn, wn_ref[...],
                        preferred_element_type=jnp.float32) + bn_ref[...]


def _update_last_body(h_ref, ag_ref, cn_ref, wuh_ref, wua_ref, bu_ref, g_ref,
                      be_ref, oh_ref):
    oh_ref[0] = _norm_update(h_ref[0], ag_ref[0], cn_ref[0], wuh_ref[...],
                             wua_ref[...], bu_ref[...], g_ref[...], be_ref[...])


def _specs():
    w = pl.BlockSpec((_H, _H), lambda b, i: (0, 0))
    v = pl.BlockSpec((1, _H), lambda b, i: (0, 0))
    hs = pl.BlockSpec((1, _BN, _H), lambda b, i: (b, i, 0))
    cs = pl.BlockSpec((1, _BN, 1), lambda b, i: (b, i, 0))
    return w, v, hs, cs


def _update_next(h, agg, cnt, wuh, wua, bu, g, beta, wn, bn):
    w, v, hs, cs = _specs()
    return pl.pallas_call(
        _update_next_body,
        grid=(_B, _N // _BN),
        in_specs=[hs, hs, cs, w, w, v, v, v, w, v],
        out_specs=[hs, hs],
        out_shape=[jax.ShapeDtypeStruct((_B, _N, _H), jnp.float32),
                   jax.ShapeDtypeStruct((_B, _N, _H), jnp.float32)],
    )(h, agg, cnt, wuh, wua, bu, g, beta, wn, bn)


def _update_last(h, agg, cnt, wuh, wua, bu, g, beta):
    w, v, hs, cs = _specs()
    return pl.pallas_call(
        _update_last_body,
        grid=(_B, _N // _BN),
        in_specs=[hs, hs, cs, w, w, v, v, v],
        out_specs=hs,
        out_shape=jax.ShapeDtypeStruct((_B, _N, _H), jnp.float32),
    )(h, agg, cnt, wuh, wua, bu, g, beta)


def kernel(h, edge_features, edge_index, n_edges, bm_w, bm_b, bu_w, bu_b,
           bn_g, bn_b, fm_w, fm_b, fu_w, fu_b, fn_g, fn_b):
    src = edge_index[:, 0, :].reshape(_B, _CPB, _CHUNK)
    dst = edge_index[:, 1, :].reshape(_B, _CPB, _CHUNK)
    ef0 = edge_features[:, :, 0].reshape(_B, _CPB, _CHUNK)
    ef1 = edge_features[:, :, 1].reshape(_B, _CPB, _CHUNK)
    meta = jnp.stack([src, dst], axis=2).reshape(-1)
    efm = jnp.stack([ef0, ef1], axis=2).reshape(-1)
    ne8 = jnp.repeat(n_edges[:, 0], 16)

    hw0 = _premsg(h, bm_w[:, :_H].T, bm_b.reshape(1, _H))
    agg0, cnt0 = _edge_back(hw0.reshape(_B * _N, _H), meta, efm,
                            ne8, bm_w[:, _H:].T)
    h1, hw1 = _update_next(h, agg0, cnt0.reshape(_B, _ROWS_SP, 1),
                           bu_w[:, :_H].T, bu_w[:, _H:].T,
                           bu_b.reshape(1, _H), bn_g.reshape(1, _H),
                           bn_b.reshape(1, _H), fm_w[:, :_H].T,
                           fm_b.reshape(1, _H))
    agg1, cnt1 = _edge_fwd(hw1.reshape(_B * _N, _H), meta, efm,
                           ne8, fm_w[:, _H:].T)
    return _update_last(h1, agg1, cnt1.reshape(_B, _ROWS_SP, 1),
                        fu_w[:, :_H].T, fu_w[:, _H:].T,
                        fu_b.reshape(1, _H), fn_g.reshape(1, _H),
                        fn_b.reshape(1, _H))


# speedup vs baseline: 10070.5068x; 1.4089x over previous
import functools

import jax
import jax.numpy as jnp
from jax import lax
from jax.experimental import pallas as pl
from jax.experimental.pallas import tpu as pltpu
from jax.experimental.pallas import tpu_sc as plsc

_B, _N, _E, _H, _ED = 4, 10000, 160000, 128, 2
_CHUNK = 128
_NTILES = 16
_NCORES = 2
_ROWS_SP = 10112
_STRIPE = _ROWS_SP // _NTILES
_CPB = _E // _CHUNK
_GARBAGE = _N


@functools.cache
def _make_edge_pass(is_back):
    mesh = plsc.VectorSubcoreMesh(core_axis_name="c", subcore_axis_name="s")

    @functools.partial(
        pl.kernel,
        out_type=(jax.ShapeDtypeStruct((_B, _ROWS_SP, _H), jnp.float32),
                  jax.ShapeDtypeStruct((_B * _ROWS_SP,), jnp.float32)),
        mesh=mesh,
        scratch_types=[
            [pltpu.VMEM((2 * _CHUNK,), jnp.int32)] * 2,
            [pltpu.VMEM((2 * _CHUNK,), jnp.float32)] * 2,
            [pltpu.VMEM((_CHUNK,), jnp.int32)] * 2,
            [pltpu.VMEM((_CHUNK,), jnp.int32)] * 2,
            [pltpu.VMEM((_CHUNK, _H), jnp.float32)] * 2,
            pltpu.VMEM((_ED, _H), jnp.float32),
            pltpu.VMEM((16 * _B,), jnp.int32),
            pltpu.VMEM((_CHUNK,), jnp.float32),
            pltpu.VMEM((_STRIPE + 8,), jnp.float32),
            pltpu.VMEM((_STRIPE,), jnp.float32),
            pltpu.VMEM_SHARED((_ROWS_SP, _H), jnp.float32),
            pltpu.VMEM_SHARED((_ROWS_SP,), jnp.float32),
            [pltpu.SemaphoreType.DMA] * 4,
            pltpu.SemaphoreType.DMA,
        ],
    )
    def edge_pass(hw_hbm, meta_hbm, efm_hbm, ne_hbm, w01_hbm,
                  agg_hbm, cnt_hbm,
                  meta_ab, efm_ab, gidx_ab, sidx_ab, grow_ab, w_v, ne_v,
                  ones_v, zline_v, cstage_v, agg_sp, cnt_sp, sems, sem):
        cid = lax.axis_index("c")
        sid = lax.axis_index("s")
        pltpu.sync_copy(ne_hbm, ne_v)
        pltpu.sync_copy(w01_hbm, w_v)
        w0 = [w_v[0, pl.ds(16 * k, 16)] for k in range(8)]
        w1 = [w_v[1, pl.ds(16 * k, 16)] for k in range(8)]
        iota = lax.broadcasted_iota(jnp.int32, (16,), 0)
        zf = jnp.zeros((16,), jnp.float32)
        onef = jnp.full((16,), 1.0, jnp.float32)
        n_c = jnp.where(sid < (_CPB % _NTILES),
                        _CPB // _NTILES + 1, _CPB // _NTILES)
        for k in range(_CHUNK // 16):
            ones_v[pl.ds(16 * k, 16)] = onef
        def zl(r, c):
            zline_v[pl.ds(16 * r, 16)] = zf
            return c
        lax.fori_loop(0, (_STRIPE + 8) // 16, zl, 0)
        zbase = sid * _STRIPE

        def start_meta(b, c, p):
            g = c * _NTILES + sid
            mb = (b * _CPB + g) * (2 * _CHUNK)
            pltpu.async_copy(meta_hbm.at[pl.ds(mb, 2 * _CHUNK)], meta_ab[p],
                             sems[p])
            pltpu.async_copy(efm_hbm.at[pl.ds(mb, 2 * _CHUNK)], efm_ab[p],
                             sems[p])

        def wait_meta(p):
            pltpu.make_async_copy(meta_hbm.at[pl.ds(0, 2 * _CHUNK)],
                                  meta_ab[p], sems[p]).wait()
            pltpu.make_async_copy(efm_hbm.at[pl.ds(0, 2 * _CHUNK)],
                                  efm_ab[p], sems[p]).wait()

        def prep(b, c, nev, p):
            e0 = (c * _NTILES + sid) * _CHUNK
            for j in range(_CHUNK // 16):
                s16 = meta_ab[p][pl.ds(16 * j, 16)]
                d16 = meta_ab[p][pl.ds(_CHUNK + 16 * j, 16)]
                eid = e0 + 16 * j + iota
                cmp = (s16 > d16) if is_back else (s16 < d16)
                m = (eid < nev) & cmp
                sidx_ab[p][pl.ds(16 * j, 16)] = jnp.where(m, d16, _GARBAGE)
                gidx_ab[p][pl.ds(16 * j, 16)] = s16 + b * _N

        def start_gather(p):
            pltpu.async_copy(hw_hbm.at[gidx_ab[p]], grow_ab[p], sems[2 + p])

        def wait_gather(p):
            pltpu.make_async_copy(hw_hbm.at[gidx_ab[p]], grow_ab[p],
                                  sems[2 + p]).wait()

        def compute(p):
            def egroup(j, cc):
                ef0g = efm_ab[p][pl.ds(16 * j, 16)]
                ef1g = efm_ab[p][pl.ds(_CHUNK + 16 * j, 16)]
                for l in range(16):
                    e = 16 * j + l
                    ef0 = jnp.full((16,), ef0g[l], jnp.float32)
                    ef1 = jnp.full((16,), ef1g[l], jnp.float32)
                    for k in range(8):
                        gk = grow_ab[p][e, pl.ds(16 * k, 16)]
                        grow_ab[p][e, pl.ds(16 * k, 16)] = jnp.maximum(
                            gk + ef0 * w0[k] + ef1 * w1[k], 0.0)
                return cc
            lax.fori_loop(0, _CHUNK // 16, egroup, 0)

        def scatter(p):
            pltpu.sync_copy(grow_ab[p], agg_sp.at[sidx_ab[p]], add=True)
            pltpu.sync_copy(ones_v, cnt_sp.at[sidx_ab[p]], add=True)

        for jj in range(_B // _NCORES):
            b = cid * (_B // _NCORES) + jj
            nev = ne_v[pl.ds(b * 16, 16)]
            g_act = (nev[0] + (_CHUNK - 1)) // _CHUNK
            n_act = jnp.clip((g_act - sid + (_NTILES - 1)) // _NTILES, 0, n_c)

            def zrow(r, c):
                for k in range(_H // 16):
                    grow_ab[0][r, pl.ds(16 * k, 16)] = zf
                return c
            lax.fori_loop(0, _CHUNK, zrow, 0)
            for part in range(4):
                pltpu.sync_copy(grow_ab[0],
                                agg_sp.at[pl.ds(zbase + part * _CHUNK, _CHUNK)])
            pltpu.sync_copy(grow_ab[0].at[pl.ds(0, _STRIPE - 4 * _CHUNK)],
                            agg_sp.at[pl.ds(zbase + 4 * _CHUNK, _STRIPE - 4 * _CHUNK)])
            pltpu.sync_copy(zline_v.at[pl.ds(0, _STRIPE)],
                            cnt_sp.at[pl.ds(zbase, _STRIPE)])
            plsc.subcore_barrier()

            @pl.when(n_act >= 1)
            def _prologue():
                start_meta(b, 0, 0)
                @pl.when(n_act >= 2)
                def _():
                    start_meta(b, 1, 1)
                wait_meta(0)
                prep(b, 0, nev, 0)
                start_gather(0)

            def pair_body(t, carry):
                ca, cb = 2 * t, 2 * t + 1
                wait_gather(0)
                @pl.when(cb < n_act)
                def _():
                    wait_meta(1)
                    prep(b, cb, nev, 1)
                    start_gather(1)
                compute(0)
                scatter(0)
                @pl.when(cb + 1 < n_act)
                def _():
                    start_meta(b, cb + 1, 0)
                @pl.when(cb < n_act)
                def _():
                    wait_gather(1)
                    @pl.when(cb + 1 < n_act)
                    def _():
                        wait_meta(0)
                        prep(b, cb + 1, nev, 0)
                        start_gather(0)
                    compute(1)
                    scatter(1)
                    @pl.when(cb + 2 < n_act)
                    def _():
                        start_meta(b, cb + 2, 1)
                return carry
            lax.fori_loop(0, (n_act + 1) // 2, pair_body, 0)
            plsc.subcore_barrier()

            pltpu.sync_copy(agg_sp.at[pl.ds(zbase, _STRIPE)],
                            agg_hbm.at[b, pl.ds(zbase, _STRIPE)])
            pltpu.sync_copy(cnt_sp.at[pl.ds(zbase, _STRIPE)], cstage_v)
            pltpu.sync_copy(cstage_v,
                            cnt_hbm.at[pl.ds(b * _ROWS_SP + zbase, _STRIPE)])
            plsc.subcore_barrier()

    return edge_pass


def _edge_back(*args):
    return _make_edge_pass(True)(*args)


def _edge_fwd(*args):
    return _make_edge_pass(False)(*args)


_BN = 1000


def _premsg_body(h_ref, w_ref, b_ref, o_ref):
    o_ref[0] = jnp.dot(h_ref[0], w_ref[...],
                       preferred_element_type=jnp.float32) + b_ref[...]


def _premsg(h, wt, bias):
    return pl.pallas_call(
        _premsg_body,
        grid=(_B, _N // _BN),
        in_specs=[
            pl.BlockSpec((1, _BN, _H), lambda b, i: (b, i, 0)),
            pl.BlockSpec((_H, _H), lambda b, i: (0, 0)),
            pl.BlockSpec((1, _H), lambda b, i: (0, 0)),
        ],
        out_specs=pl.BlockSpec((1, _BN, _H), lambda b, i: (b, i, 0)),
        out_shape=jax.ShapeDtypeStruct((_B, _N, _H), jnp.float32),
    )(h, wt, bias)


def _norm_update(h, agg, cnt, wuh, wua, bu, g, beta):
    aggn = agg / jnp.maximum(cnt, 1.0)
    u = (jnp.dot(h, wuh, preferred_element_type=jnp.float32)
         + jnp.dot(aggn, wua, preferred_element_type=jnp.float32) + bu)
    x = h + jnp.maximum(u, 0.0)
    mu = jnp.mean(x, axis=-1, keepdims=True)
    xc = x - mu
    var = jnp.mean(xc * xc, axis=-1, keepdims=True)
    return xc * lax.rsqrt(var + 1e-5) * g + beta


def _update_next_body(h_ref, ag_ref, cn_ref, wuh_ref, wua_ref, bu_ref, g_ref,
                      be_ref, wn_ref, bn_ref, oh_ref, ow_ref):
    hn = _norm_update(h_ref[0], ag_ref[0], cn_ref[0], wuh_ref[...],
                      wua_ref[...], bu_ref[...], g_ref[...], be_ref[...])
    oh_ref[0] = hn
    ow_ref[0] = jnp.dot(hn, wn_ref[...],
                        preferred_element_type=jnp.float32) + bn_ref[...]


def _update_last_body(h_ref, ag_ref, cn_ref, wuh_ref, wua_ref, bu_ref, g_ref,
                      be_ref, oh_ref):
    oh_ref[0] = _norm_update(h_ref[0], ag_ref[0], cn_ref[0], wuh_ref[...],
                             wua_ref[...], bu_ref[...], g_ref[...], be_ref[...])


def _specs():
    w = pl.BlockSpec((_H, _H), lambda b, i: (0, 0))
    v = pl.BlockSpec((1, _H), lambda b, i: (0, 0))
    hs = pl.BlockSpec((1, _BN, _H), lambda b, i: (b, i, 0))
    cs = pl.BlockSpec((1, _BN, 1), lambda b, i: (b, i, 0))
    return w, v, hs, cs


def _update_next(h, agg, cnt, wuh, wua, bu, g, beta, wn, bn):
    w, v, hs, cs = _specs()
    return pl.pallas_call(
        _update_next_body,
        grid=(_B, _N // _BN),
        in_specs=[hs, hs, cs, w, w, v, v, v, w, v],
        out_specs=[hs, hs],
        out_shape=[jax.ShapeDtypeStruct((_B, _N, _H), jnp.float32),
                   jax.ShapeDtypeStruct((_B, _N, _H), jnp.float32)],
    )(h, agg, cnt, wuh, wua, bu, g, beta, wn, bn)


def _update_last(h, agg, cnt, wuh, wua, bu, g, beta):
    w, v, hs, cs = _specs()
    return pl.pallas_call(
        _update_last_body,
        grid=(_B, _N // _BN),
        in_specs=[hs, hs, cs, w, w, v, v, v],
        out_specs=hs,
        out_shape=jax.ShapeDtypeStruct((_B, _N, _H), jnp.float32),
    )(h, agg, cnt, wuh, wua, bu, g, beta)


def kernel(h, edge_features, edge_index, n_edges, bm_w, bm_b, bu_w, bu_b,
           bn_g, bn_b, fm_w, fm_b, fu_w, fu_b, fn_g, fn_b):
    src = edge_index[:, 0, :].reshape(_B, _CPB, _CHUNK)
    dst = edge_index[:, 1, :].reshape(_B, _CPB, _CHUNK)
    ef0 = edge_features[:, :, 0].reshape(_B, _CPB, _CHUNK)
    ef1 = edge_features[:, :, 1].reshape(_B, _CPB, _CHUNK)
    meta = jnp.stack([src, dst], axis=2).reshape(-1)
    efm = jnp.stack([ef0, ef1], axis=2).reshape(-1)
    ne8 = jnp.repeat(n_edges[:, 0], 16)

    hw0 = _premsg(h, bm_w[:, :_H].T, bm_b.reshape(1, _H))
    agg0, cnt0 = _edge_back(hw0.reshape(_B * _N, _H), meta, efm,
                            ne8, bm_w[:, _H:].T)
    h1, hw1 = _update_next(h, agg0, cnt0.reshape(_B, _ROWS_SP, 1),
                           bu_w[:, :_H].T, bu_w[:, _H:].T,
                           bu_b.reshape(1, _H), bn_g.reshape(1, _H),
                           bn_b.reshape(1, _H), fm_w[:, :_H].T,
                           fm_b.reshape(1, _H))
    agg1, cnt1 = _edge_fwd(hw1.reshape(_B * _N, _H), meta, efm,
                           ne8, fm_w[:, _H:].T)
    return _update_last(h1, agg1, cnt1.reshape(_B, _ROWS_SP, 1),
                        fu_w[:, :_H].T, fu_w[:, _H:].T,
                        fu_b.reshape(1, _H), fn_g.reshape(1, _H),
                        fn_b.reshape(1, _H))
